# baseline (speedup 1.0000x reference)
import functools

import jax
import jax.numpy as jnp
from jax import lax
from jax.experimental import pallas as pl
from jax.experimental.pallas import tpu as pltpu
from jax.experimental.pallas import tpu_sc as plsc

N_NODES = 100000
EMB_DIM = 64
D_FEAT = 128
OUT_DIM = EMB_DIM + D_FEAT

NC = 2
NS = 16
NW = NC * NS

S = 160
NBLK = N_NODES // S
NBLK_W = (NBLK + NW - 1) // NW


def _body(ids_h, feats, table, out,
          idx0, idx1, cb0, cb1,
          isem0, isem1, rsem0, rsem1, fsem0, fsem1, wsem0, wsem1):
    wid = lax.axis_index("s") * NC + lax.axis_index("c")
    idx = (idx0, idx1)
    cb = (cb0, cb1)
    isem = (isem0, isem1)
    rsem = (rsem0, rsem1)
    fsem = (fsem0, fsem1)
    wsem = (wsem0, wsem1)

    def base_of(i):
        k = wid + i * NW
        k = jnp.where(k < NBLK, k, k - NW)
        return k * S

    ih = [None]
    wbh = {}

    for i in range(NBLK_W):
        p = i % 2
        base = base_of(i)
        if i >= 2:
            wbh[i - 2].wait()
        if i == 0:
            pltpu.sync_copy(ids_h.at[pl.ds(base, S)], idx[p])
        else:
            ih[0].wait()
        fh = pltpu.async_copy(
            feats.at[pl.ds(base, S), :],
            cb[p].at[:, pl.ds(EMB_DIM, D_FEAT)],
            fsem[p],
        )

        def row_chunk(c, _, p=p):
            vals = idx[p][pl.ds(c * 16, 16)]
            for lane in range(16):
                rr = vals[lane]
                pltpu.async_copy(
                    table.at[pl.ds(rr, 1), :],
                    cb[p].at[pl.ds(c * 16 + lane, 1), pl.ds(0, EMB_DIM)],
                    rsem[p],
                )
            return 0

        lax.fori_loop(0, S // 16, row_chunk, 0)

        if i + 1 < NBLK_W:
            ih[0] = pltpu.async_copy(
                ids_h.at[pl.ds(base_of(i + 1), S)],
                idx[(i + 1) % 2], isem[(i + 1) % 2])

        pltpu.make_async_copy(
            table.at[pl.ds(0, S), :],
            cb[p].at[:, pl.ds(0, EMB_DIM)],
            rsem[p],
        ).wait()
        fh.wait()
        wbh[i] = pltpu.async_copy(
            cb[p], out.at[pl.ds(base, S), :], wsem[p])

    wbh[NBLK_W - 2].wait()
    wbh[NBLK_W - 1].wait()


@jax.jit
def _run(ids, feats, table):
    mesh = plsc.VectorSubcoreMesh(core_axis_name="c", subcore_axis_name="s")
    return pl.kernel(
        _body,
        mesh=mesh,
        compiler_params=pltpu.CompilerParams(use_tc_tiling_on_sc=False),
        out_type=jax.ShapeDtypeStruct((N_NODES, OUT_DIM), jnp.float32),
        scratch_types=[
            pltpu.VMEM((S,), jnp.int32),
            pltpu.VMEM((S,), jnp.int32),
            pltpu.VMEM((S, OUT_DIM), jnp.float32),
            pltpu.VMEM((S, OUT_DIM), jnp.float32),
            pltpu.SemaphoreType.DMA,
            pltpu.SemaphoreType.DMA,
            pltpu.SemaphoreType.DMA,
            pltpu.SemaphoreType.DMA,
            pltpu.SemaphoreType.DMA,
            pltpu.SemaphoreType.DMA,
            pltpu.SemaphoreType.DMA,
            pltpu.SemaphoreType.DMA,
        ],
    )(ids, feats, table)


def kernel(ids, feats, table):
    return _run(ids.astype(jnp.int32), feats, table)


# speedup vs baseline: 1.4969x; 1.4969x over previous
import functools

import jax
import jax.numpy as jnp
from jax import lax
from jax.experimental import pallas as pl
from jax.experimental.pallas import tpu as pltpu
from jax.experimental.pallas import tpu_sc as plsc

N_NODES = 100000
EMB_DIM = 64
D_FEAT = 128
OUT_DIM = EMB_DIM + D_FEAT

NC = 2
NS = 16
NW = NC * NS

S = 80
R = S // 16
NBLK = N_NODES // S
NBLK_W = (NBLK + NW - 1) // NW
NPAIR = NBLK_W // 2


def _body(ids_h, feats, table, out,
          idx0, idx1, fb0, fb1, cb0, cb1,
          ts00, ts01, ts02, ts03, ts04, ts05, ts06, ts07,
          ts08, ts09, ts10, ts11, ts12, ts13, ts14, ts15,
          ts16, ts17, ts18, ts19, ts20, ts21, ts22, ts23,
          ts24, ts25, ts26, ts27, ts28, ts29, ts30, ts31,
          isem0, isem1, fsem0, fsem1, wsem0, wsem1, tsem0, tsem1):
    wid = lax.axis_index("s") * NC + lax.axis_index("c")
    idx = (idx0, idx1)
    fb = (fb0, fb1)
    cb = (cb0, cb1)
    tslot = ((ts00, ts01, ts02, ts03, ts04, ts05, ts06, ts07,
              ts08, ts09, ts10, ts11, ts12, ts13, ts14, ts15),
             (ts16, ts17, ts18, ts19, ts20, ts21, ts22, ts23,
              ts24, ts25, ts26, ts27, ts28, ts29, ts30, ts31))
    isem = (isem0, isem1)
    fsem = (fsem0, fsem1)
    wsem = (wsem0, wsem1)
    tsem = (tsem0, tsem1)

    def base_of(i):
        k = wid + i * NW
        k = jnp.where(k < NBLK, k, k - NW)
        return k * S

    def drain_ids(p):
        pltpu.make_async_copy(ids_h.at[pl.ds(0, S)], idx[p], isem[p]).wait()

    def drain_feats(p):
        pltpu.make_async_copy(feats.at[pl.ds(0, S), :], fb[p], fsem[p]).wait()

    def drain_wb(p):
        pltpu.make_async_copy(out.at[pl.ds(0, S), :], cb[p], wsem[p]).wait()

    def drain_tiles(ring):
        for lane in range(16):
            pltpu.make_async_copy(
                table.at[pl.ds(0, 8), :], tslot[ring][lane], tsem[ring]).wait()

    def fire_round(vals, ring, p):
        for lane in range(16):
            rr = vals[lane]
            t8 = pl.multiple_of((rr // 8) * 8, 8)
            pltpu.async_copy(
                table.at[pl.ds(t8, 8), :], tslot[ring][lane], tsem[ring])

    def select_round(vals, ring, q, p):
        for lane in range(16):
            rr = vals[lane]
            srow = rr % 8
            row = q * 16 + lane
            for c in range(0, EMB_DIM, 16):
                cb[p][row, pl.ds(c, 16)] = tslot[ring][lane][srow, pl.ds(c, 16)]

    def block(i, j, p):
        base = base_of(i)

        @pl.when(j > 0)
        def _():
            drain_wb(p)

        drain_ids(p)
        pltpu.async_copy(feats.at[pl.ds(base, S), :], fb[p], fsem[p])
        pltpu.async_copy(
            ids_h.at[pl.ds(base_of(i + 1), S)], idx[1 - p], isem[1 - p])

        vals_prev = None
        for q in range(R):
            vals = idx[p][pl.ds(q * 16, 16)]
            fire_round(vals, q % 2, p)
            if q > 0:
                drain_tiles((q - 1) % 2)
                select_round(vals_prev, (q - 1) % 2, q - 1, p)
            vals_prev = vals
        drain_tiles((R - 1) % 2)
        select_round(vals_prev, (R - 1) % 2, R - 1, p)

        drain_feats(p)

        def fcopy(r, _):
            for c in range(0, D_FEAT, 16):
                cb[p][r, pl.ds(EMB_DIM + c, 16)] = fb[p][r, pl.ds(c, 16)]
            return 0

        lax.fori_loop(0, S, fcopy, 0)

        pltpu.async_copy(cb[p], out.at[pl.ds(base, S), :], wsem[p])

    pltpu.async_copy(ids_h.at[pl.ds(base_of(0), S)], idx[0], isem[0])

    def pair(j, _):
        block(2 * j, j, 0)
        block(2 * j + 1, j, 1)
        return 0

    lax.fori_loop(0, NPAIR, pair, 0)

    drain_wb(0)
    drain_wb(1)
    drain_ids(0)


@jax.jit
def _run(ids, feats, table):
    mesh = plsc.VectorSubcoreMesh(core_axis_name="c", subcore_axis_name="s")
    tile_types = [pltpu.VMEM((8, EMB_DIM), jnp.float32)] * 32
    return pl.kernel(
        _body,
        mesh=mesh,
        out_type=jax.ShapeDtypeStruct((N_NODES, OUT_DIM), jnp.float32),
        scratch_types=[
            pltpu.VMEM((S,), jnp.int32),
            pltpu.VMEM((S,), jnp.int32),
            pltpu.VMEM((S, D_FEAT), jnp.float32),
            pltpu.VMEM((S, D_FEAT), jnp.float32),
            pltpu.VMEM((S, OUT_DIM), jnp.float32),
            pltpu.VMEM((S, OUT_DIM), jnp.float32),
        ] + tile_types + [
            pltpu.SemaphoreType.DMA,
            pltpu.SemaphoreType.DMA,
            pltpu.SemaphoreType.DMA,
            pltpu.SemaphoreType.DMA,
            pltpu.SemaphoreType.DMA,
            pltpu.SemaphoreType.DMA,
            pltpu.SemaphoreType.DMA,
            pltpu.SemaphoreType.DMA,
        ],
    )(ids, feats, table)


def kernel(ids, feats, table):
    return _run(ids.astype(jnp.int32), feats, table)


# speedup vs baseline: 1.5147x; 1.0119x over previous
import functools

import jax
import jax.numpy as jnp
from jax import lax
from jax.experimental import pallas as pl
from jax.experimental.pallas import tpu as pltpu
from jax.experimental.pallas import tpu_sc as plsc

N_NODES = 100000
EMB_DIM = 64
D_FEAT = 128
OUT_DIM = EMB_DIM + D_FEAT

NC = 2
NS = 16
NW = NC * NS

S = 80
R = S // 16
NBLK = N_NODES // S
NBLK_W = (NBLK + NW - 1) // NW
NPAIR = NBLK_W // 2


def _body(ids_h, feats, table, out,
          idx0, idx1, fb0, fb1, cb0, cb1,
          ts00, ts01, ts02, ts03, ts04, ts05, ts06, ts07,
          ts08, ts09, ts10, ts11, ts12, ts13, ts14, ts15,
          ts16, ts17, ts18, ts19, ts20, ts21, ts22, ts23,
          ts24, ts25, ts26, ts27, ts28, ts29, ts30, ts31,
          isem0, isem1, fsem0, fsem1, wsem0, wsem1, tsem0, tsem1):
    wid = lax.axis_index("s") * NC + lax.axis_index("c")
    idx = (idx0, idx1)
    fb = (fb0, fb1)
    cb = (cb0, cb1)
    tslot = ((ts00, ts01, ts02, ts03, ts04, ts05, ts06, ts07,
              ts08, ts09, ts10, ts11, ts12, ts13, ts14, ts15),
             (ts16, ts17, ts18, ts19, ts20, ts21, ts22, ts23,
              ts24, ts25, ts26, ts27, ts28, ts29, ts30, ts31))
    isem = (isem0, isem1)
    fsem = (fsem0, fsem1)
    wsem = (wsem0, wsem1)
    tsem = (tsem0, tsem1)

    def base_of(i):
        k = wid + i * NW
        k = jnp.where(k < NBLK, k, k - NW)
        return k * S

    def drain_ids(p):
        pltpu.make_async_copy(ids_h.at[pl.ds(0, S)], idx[p], isem[p]).wait()

    def drain_feats(p):
        pltpu.make_async_copy(feats.at[pl.ds(0, S), :], fb[p], fsem[p]).wait()

    def drain_wb(p):
        pltpu.make_async_copy(out.at[pl.ds(0, S), :], cb[p], wsem[p]).wait()

    def drain_tiles(ring):
        for lane in range(16):
            pltpu.make_async_copy(
                table.at[pl.ds(0, 8), :], tslot[ring][lane], tsem[ring]).wait()

    def fire_round(vals, ring, p):
        for lane in range(16):
            rr = vals[lane]
            t8 = pl.multiple_of((rr // 8) * 8, 8)
            pltpu.async_copy(
                table.at[pl.ds(t8, 8), :], tslot[ring][lane], tsem[ring])

    def select_round(vals, ring, q, p):
        for lane in range(16):
            rr = vals[lane]
            srow = rr % 8
            row = q * 16 + lane
            for c in range(0, EMB_DIM, 16):
                cb[p][row, pl.ds(c, 16)] = tslot[ring][lane][srow, pl.ds(c, 16)]

    def block(i, j, p):
        base = base_of(i)

        @pl.when(j > 0)
        def _():
            drain_wb(p)

        drain_ids(p)
        pltpu.async_copy(feats.at[pl.ds(base, S), :], fb[p], fsem[p])
        pltpu.async_copy(
            ids_h.at[pl.ds(base_of(i + 1), S)], idx[1 - p], isem[1 - p])

        vals_prev = None
        for q in range(R):
            vals = idx[p][pl.ds(q * 16, 16)]
            fire_round(vals, q % 2, p)
            if q > 0:
                drain_tiles((q - 1) % 2)
                select_round(vals_prev, (q - 1) % 2, q - 1, p)
            vals_prev = vals
        drain_tiles((R - 1) % 2)
        select_round(vals_prev, (R - 1) % 2, R - 1, p)

        drain_feats(p)
        for r in range(S):
            for c in range(0, D_FEAT, 16):
                cb[p][r, pl.ds(EMB_DIM + c, 16)] = fb[p][r, pl.ds(c, 16)]

        pltpu.async_copy(cb[p], out.at[pl.ds(base, S), :], wsem[p])

    pltpu.async_copy(ids_h.at[pl.ds(base_of(0), S)], idx[0], isem[0])

    def pair(j, _):
        block(2 * j, j, 0)
        block(2 * j + 1, j, 1)
        return 0

    lax.fori_loop(0, NPAIR, pair, 0)

    drain_wb(0)
    drain_wb(1)
    drain_ids(0)


@jax.jit
def _run(ids, feats, table):
    mesh = plsc.VectorSubcoreMesh(core_axis_name="c", subcore_axis_name="s")
    tile_types = [pltpu.VMEM((8, EMB_DIM), jnp.float32)] * 32
    return pl.kernel(
        _body,
        mesh=mesh,
        out_type=jax.ShapeDtypeStruct((N_NODES, OUT_DIM), jnp.float32),
        scratch_types=[
            pltpu.VMEM((S,), jnp.int32),
            pltpu.VMEM((S,), jnp.int32),
            pltpu.VMEM((S, D_FEAT), jnp.float32),
            pltpu.VMEM((S, D_FEAT), jnp.float32),
            pltpu.VMEM((S, OUT_DIM), jnp.float32),
            pltpu.VMEM((S, OUT_DIM), jnp.float32),
        ] + tile_types + [
            pltpu.SemaphoreType.DMA,
            pltpu.SemaphoreType.DMA,
            pltpu.SemaphoreType.DMA,
            pltpu.SemaphoreType.DMA,
            pltpu.SemaphoreType.DMA,
            pltpu.SemaphoreType.DMA,
            pltpu.SemaphoreType.DMA,
            pltpu.SemaphoreType.DMA,
        ],
    )(ids, feats, table)


def kernel(ids, feats, table):
    return _run(ids.astype(jnp.int32), feats, table)


# speedup vs baseline: 1.5670x; 1.0346x over previous
import functools

import jax
import jax.numpy as jnp
from jax import lax
from jax.experimental import pallas as pl
from jax.experimental.pallas import tpu as pltpu
from jax.experimental.pallas import tpu_sc as plsc

N_NODES = 100000
EMB_DIM = 64
D_FEAT = 128
OUT_DIM = EMB_DIM + D_FEAT

NC = 2
NS = 16
NW = NC * NS

S = 80
R = S // 16
NRING = 4
NBLK = N_NODES // S
NBLK_W = (NBLK + NW - 1) // NW
NPAIR = NBLK_W // 2


def _body(ids_h, feats, table, out, *rest):
    wid = lax.axis_index("s") * NC + lax.axis_index("c")
    idx = rest[0:2]
    fb = rest[2:4]
    cb = rest[4:6]
    slots = [rest[6 + 16 * r: 6 + 16 * (r + 1)] for r in range(NRING)]
    nslot = 6 + 16 * NRING
    isem = rest[nslot: nslot + 2]
    fsem = rest[nslot + 2: nslot + 4]
    wsem = rest[nslot + 4: nslot + 6]
    tsem = rest[nslot + 6: nslot + 6 + NRING]

    def base_of(i):
        k = wid + i * NW
        k = jnp.where(k < NBLK, k, k - NW)
        return k * S

    def drain_ids(p):
        pltpu.make_async_copy(ids_h.at[pl.ds(0, S)], idx[p], isem[p]).wait()

    def drain_feats(p):
        pltpu.make_async_copy(feats.at[pl.ds(0, S), :], fb[p], fsem[p]).wait()

    def drain_wb(p):
        pltpu.make_async_copy(out.at[pl.ds(0, S), :], cb[p], wsem[p]).wait()

    def drain_ring(ring):
        for lane in range(16):
            pltpu.make_async_copy(
                table.at[pl.ds(0, 8), :], slots[ring][lane], tsem[ring]).wait()

    def fire_round(vals, ring):
        for lane in range(16):
            rr = vals[lane]
            t8 = pl.multiple_of((rr // 8) * 8, 8)
            pltpu.async_copy(
                table.at[pl.ds(t8, 8), :], slots[ring][lane], tsem[ring])

    def select_round(vals, ring, q, p):
        for lane in range(16):
            srow = vals[lane] % 8
            row = q * 16 + lane
            for c in range(0, EMB_DIM, 16):
                cb[p][row, pl.ds(c, 16)] = slots[ring][lane][srow, pl.ds(c, 16)]

    def fcopy_chunk(p, chunk):
        for r in range(chunk * 16, chunk * 16 + 16):
            for c in range(0, D_FEAT, 16):
                cb[p][r, pl.ds(EMB_DIM + c, 16)] = fb[p][r, pl.ds(c, 16)]

    def block(i, j, p):
        base = base_of(i)

        @pl.when(j > 0)
        def _():
            drain_wb(p)

        drain_ids(p)
        pltpu.async_copy(feats.at[pl.ds(base, S), :], fb[p], fsem[p])
        pltpu.async_copy(
            ids_h.at[pl.ds(base_of(i + 1), S)], idx[1 - p], isem[1 - p])

        vals = [idx[p][pl.ds(q * 16, 16)] for q in range(R)]
        fire_round(vals[0], 0)
        fire_round(vals[1], 1)
        drain_feats(p)
        for q in range(R):
            fcopy_chunk(p, q)
            drain_ring(q % NRING)
            select_round(vals[q], q % NRING, q, p)
            if q + 2 < R:
                fire_round(vals[q + 2], (q + 2) % NRING)

        pltpu.async_copy(cb[p], out.at[pl.ds(base, S), :], wsem[p])

    pltpu.async_copy(ids_h.at[pl.ds(base_of(0), S)], idx[0], isem[0])

    def pair(j, _):
        block(2 * j, j, 0)
        block(2 * j + 1, j, 1)
        return 0

    lax.fori_loop(0, NPAIR, pair, 0)

    drain_wb(0)
    drain_wb(1)
    drain_ids(0)


@jax.jit
def _run(ids, feats, table):
    mesh = plsc.VectorSubcoreMesh(core_axis_name="c", subcore_axis_name="s")
    return pl.kernel(
        _body,
        mesh=mesh,
        out_type=jax.ShapeDtypeStruct((N_NODES, OUT_DIM), jnp.float32),
        scratch_types=[
            pltpu.VMEM((S,), jnp.int32),
            pltpu.VMEM((S,), jnp.int32),
            pltpu.VMEM((S, D_FEAT), jnp.float32),
            pltpu.VMEM((S, D_FEAT), jnp.float32),
            pltpu.VMEM((S, OUT_DIM), jnp.float32),
            pltpu.VMEM((S, OUT_DIM), jnp.float32),
        ] + [pltpu.VMEM((8, EMB_DIM), jnp.float32)] * (16 * NRING) + [
            pltpu.SemaphoreType.DMA,
            pltpu.SemaphoreType.DMA,
            pltpu.SemaphoreType.DMA,
            pltpu.SemaphoreType.DMA,
            pltpu.SemaphoreType.DMA,
            pltpu.SemaphoreType.DMA,
        ] + [pltpu.SemaphoreType.DMA] * NRING,
    )(ids, feats, table)


def kernel(ids, feats, table):
    return _run(ids.astype(jnp.int32), feats, table)


# speedup vs baseline: 1.6106x; 1.0278x over previous
import functools

import jax
import jax.numpy as jnp
from jax import lax
from jax.experimental import pallas as pl
from jax.experimental.pallas import tpu as pltpu
from jax.experimental.pallas import tpu_sc as plsc

N_NODES = 100000
EMB_DIM = 64
D_FEAT = 128
OUT_DIM = EMB_DIM + D_FEAT

NC = 2
NS = 16
NW = NC * NS

S = 80
R = S // 16
NRING = 4
NBLK = N_NODES // S
NBLK_W = (NBLK + NW - 1) // NW
NPAIR = NBLK_W // 2


def _body(ids_h, feats, table, out, *rest):
    wid = lax.axis_index("s") * NC + lax.axis_index("c")
    idx = rest[0:2]
    fb = rest[2:4]
    cb = rest[4:6]
    dr = rest[6]
    slots = [rest[7 + 16 * r: 7 + 16 * (r + 1)] for r in range(NRING)]
    nslot = 7 + 16 * NRING
    isem = rest[nslot: nslot + 2]
    fsem = rest[nslot + 2: nslot + 4]
    wsem = rest[nslot + 4: nslot + 6]
    tsem = rest[nslot + 6: nslot + 6 + NRING]

    def base_of(i):
        k = wid + i * NW
        k = jnp.where(k < NBLK, k, k - NW)
        return k * S

    def drain_ids(p):
        pltpu.make_async_copy(ids_h.at[pl.ds(0, S)], idx[p], isem[p]).wait()

    def drain_feats(p):
        pltpu.make_async_copy(feats.at[pl.ds(0, S), :], fb[p], fsem[p]).wait()

    def drain_wb(p):
        pltpu.make_async_copy(out.at[pl.ds(0, S), :], cb[p], wsem[p]).wait()

    def drain_ring(ring):
        for lane in range(16):
            pltpu.make_async_copy(
                table.at[pl.ds(0, 8), :], slots[ring][lane], tsem[ring]).wait()

    def fire_round(vals, ring):
        for lane in range(16):
            rr = vals[lane]
            t8 = pl.multiple_of((rr // 8) * 8, 8)
            pltpu.async_copy(
                table.at[pl.ds(t8, 8), :], slots[ring][lane], tsem[ring])

    def select_round(vals, ring, q, p):
        for lane in range(16):
            srow = vals[lane] % 8
            row = q * 16 + lane
            for c in range(0, EMB_DIM, 16):
                cb[p][row, pl.ds(c, 16)] = slots[ring][lane][srow, pl.ds(c, 16)]

    def fcopy_chunk(p, chunk):
        for r in range(chunk * 16, chunk * 16 + 16):
            for c in range(0, D_FEAT, 16):
                cb[p][r, pl.ds(EMB_DIM + c, 16)] = fb[p][r, pl.ds(c, 16)]

    def block(i, j, p):
        base = base_of(i)

        @pl.when(j > 0)
        def _():
            drain_wb(p)

        drain_ids(p)
        pltpu.async_copy(feats.at[pl.ds(base, S), :], fb[p], fsem[p])
        pltpu.async_copy(
            ids_h.at[pl.ds(base_of(i + 1), S)], idx[1 - p], isem[1 - p])

        vals = [idx[p][pl.ds(q * 16, 16)] for q in range(R)]
        fire_round(vals[0], 0)
        fire_round(vals[1], 1)
        drain_feats(p)
        for q in range(R):
            fcopy_chunk(p, q)
            if q + 2 < R:
                fire_round(vals[q + 2], (q + 2) % NRING)
            drain_ring(q % NRING)
            select_round(vals[q], q % NRING, q, p)

        pltpu.async_copy(cb[p], out.at[pl.ds(base, S), :], wsem[p])

    pltpu.async_copy(ids_h.at[pl.ds(base_of(0), S)], idx[0], isem[0])

    def pair(j, _):
        block(2 * j, j, 0)
        block(2 * j + 1, j, 1)
        return 0

    lax.fori_loop(0, NPAIR, pair, 0)

    drain_wb(0)
    drain_wb(1)
    drain_ids(0)


@jax.jit
def _run(ids, feats, table):
    mesh = plsc.VectorSubcoreMesh(core_axis_name="c", subcore_axis_name="s")
    return pl.kernel(
        _body,
        mesh=mesh,
        out_type=jax.ShapeDtypeStruct((N_NODES, OUT_DIM), jnp.float32),
        scratch_types=[
            pltpu.VMEM((S,), jnp.int32),
            pltpu.VMEM((S,), jnp.int32),
            pltpu.VMEM((S, D_FEAT), jnp.float32),
            pltpu.VMEM((S, D_FEAT), jnp.float32),
            pltpu.VMEM((S, OUT_DIM), jnp.float32),
            pltpu.VMEM((S, OUT_DIM), jnp.float32),
            pltpu.VMEM((64, D_FEAT), jnp.float32),
        ] + [pltpu.VMEM((8, EMB_DIM), jnp.float32)] * (16 * NRING) + [
            pltpu.SemaphoreType.DMA,
            pltpu.SemaphoreType.DMA,
            pltpu.SemaphoreType.DMA,
            pltpu.SemaphoreType.DMA,
            pltpu.SemaphoreType.DMA,
            pltpu.SemaphoreType.DMA,
        ] + [pltpu.SemaphoreType.DMA] * NRING,
    )(ids, feats, table)


def kernel(ids, feats, table):
    return _run(ids.astype(jnp.int32), feats, table)
